# baseline (speedup 1.0000x reference)
import functools

import jax
import jax.numpy as jnp
from jax import lax
from jax.experimental import pallas as pl
from jax.experimental.pallas import tpu as pltpu
from jax.experimental.pallas import tpu_sc as plsc

VOCAB = 1000000
EMBED_DIM = 64
BATCH = 4096
SEQ = 200

NC = 2
NS = 16
NW = NC * NS

N_ROWS = BATCH * SEQ
PER_W = N_ROWS // NW
CH = 128
NCHUNK = PER_W // CH
NBUF = 4
NGROUP = NCHUNK // NBUF

N_PAIR = N_ROWS // 2
PAIR_W = N_PAIR // NW
KP = 256
NPBLK = PAIR_W // KP


@functools.partial(
    pl.kernel,
    out_type=jax.ShapeDtypeStruct((N_ROWS, EMBED_DIM), jnp.float32),
    mesh=plsc.VectorSubcoreMesh(core_axis_name="c", subcore_axis_name="s"),
    scratch_types=[
        pltpu.VMEM((NCHUNK, CH), jnp.int32),
        pltpu.VMEM((NBUF, CH, EMBED_DIM), jnp.float32),
    ] + [pltpu.SemaphoreType.DMA] * NBUF,
    compiler_params=pltpu.CompilerParams(use_tc_tiling_on_sc=False),
)
def _gather_kernel(idx_hbm, table_hbm, out2, idx_v, rows_v, *gsem):
    wid = lax.axis_index("s") * NC + lax.axis_index("c")
    base = wid * PER_W
    pltpu.sync_copy(idx_hbm.at[wid], idx_v)

    def fire(j, b):
        pltpu.async_copy(table_hbm.at[idx_v.at[j]], rows_v.at[b], gsem[b])

    def wait(b):
        pltpu.make_async_copy(out2.at[pl.ds(0, CH)], rows_v.at[b], gsem[b]).wait()

    for b in range(NBUF):
        fire(b, b)

    def group(t, carry):
        for b in range(NBUF):
            j = t * NBUF + b
            wait(b)
            pltpu.sync_copy(rows_v.at[b], out2.at[pl.ds(base + j * CH, CH)])
            fire(j + NBUF, b)
        return carry

    lax.fori_loop(0, NGROUP - 1, group, 0)

    for b in range(NBUF):
        j = (NGROUP - 1) * NBUF + b
        wait(b)
        pltpu.sync_copy(rows_v.at[b], out2.at[pl.ds(base + j * CH, CH)])


UNP_K = 512
UNP_GRID = N_PAIR // UNP_K


def _unpack_tc_body(in_ref, out_ref):
    a = in_ref[...]
    out_ref[:, 0, :] = a[:, :EMBED_DIM]
    out_ref[:, 1, :] = a[:, EMBED_DIM:]


_unpack_tc = pl.pallas_call(
    _unpack_tc_body,
    grid=(UNP_GRID,),
    in_specs=[pl.BlockSpec((UNP_K, 2 * EMBED_DIM), lambda i: (i, 0))],
    out_specs=pl.BlockSpec((UNP_K, 2, EMBED_DIM), lambda i: (i, 0, 0)),
    out_shape=jax.ShapeDtypeStruct((N_PAIR, 2, EMBED_DIM), jnp.float32),
)


def kernel(x, embedding_weight):
    idx = x.reshape(NW, NCHUNK, CH)
    packed = _gather_kernel(idx, embedding_weight)
    out = _unpack_tc(packed.reshape(N_PAIR, 2 * EMBED_DIM))
    return out.reshape(BATCH, SEQ, EMBED_DIM)


# speedup vs baseline: 1.0094x; 1.0094x over previous
import functools

import jax
import jax.numpy as jnp
from jax import lax
from jax.experimental import pallas as pl
from jax.experimental.pallas import tpu as pltpu
from jax.experimental.pallas import tpu_sc as plsc

VOCAB = 1000000
EMBED_DIM = 64
BATCH = 4096
SEQ = 200

NC = 2
NS = 16
NW = NC * NS

N_ROWS = BATCH * SEQ
PER_W = N_ROWS // NW
CH = 128
NCHUNK = PER_W // CH
NBUF = 4
NGROUP = NCHUNK // NBUF

N_PAIR = N_ROWS // 2
V_PAIR = VOCAB // 2

PK_BLK = 1250
UNP_K = 1024


def _pack_tc_body(in_ref, out_ref):
    out_ref[:, :EMBED_DIM] = in_ref[:, 0, :]
    out_ref[:, EMBED_DIM:] = in_ref[:, 1, :]


_pack_tc = pl.pallas_call(
    _pack_tc_body,
    grid=(V_PAIR // (PK_BLK * 4),),
    in_specs=[pl.BlockSpec((PK_BLK * 4, 2, EMBED_DIM), lambda i: (i, 0, 0))],
    out_specs=pl.BlockSpec((PK_BLK * 4, 2 * EMBED_DIM), lambda i: (i, 0)),
    out_shape=jax.ShapeDtypeStruct((V_PAIR, 2 * EMBED_DIM), jnp.float32),
)


@functools.partial(
    pl.kernel,
    out_type=jax.ShapeDtypeStruct((N_PAIR, 2 * EMBED_DIM), jnp.float32),
    mesh=plsc.VectorSubcoreMesh(core_axis_name="c", subcore_axis_name="s"),
    scratch_types=[
        pltpu.VMEM((NCHUNK, CH), jnp.int32),
        pltpu.VMEM((NBUF, CH), jnp.int32),
        pltpu.VMEM((NBUF, CH, 2 * EMBED_DIM), jnp.float32),
        pltpu.VMEM((NBUF, CH // 2, 2 * EMBED_DIM), jnp.float32),
    ]
    + [pltpu.SemaphoreType.DMA] * NBUF
    + [pltpu.SemaphoreType.DMA] * NBUF,
)
def _gather_kernel(idx_hbm, table_hbm, out_hbm, idx_v, hi_v, rows_v, outb_v,
                   *sems):
    gsem = sems[:NBUF]
    wsem = sems[NBUF:]
    wid = lax.axis_index("s") * NC + lax.axis_index("c")
    pbase = wid * (PER_W // 2)
    pltpu.sync_copy(idx_hbm.at[wid], idx_v)

    def compute_hi(j, b):
        for m in range(CH // 16):
            hi_v[b, pl.ds(m * 16, 16)] = lax.shift_right_logical(
                idx_v[j, pl.ds(m * 16, 16)], 1)

    def fire_gather(b):
        pltpu.async_copy(table_hbm.at[hi_v.at[b]], rows_v.at[b], gsem[b])

    def wait_gather(b):
        pltpu.make_async_copy(
            table_hbm.at[hi_v.at[b]], rows_v.at[b], gsem[b]).wait()

    def fire_write(j, b):
        pltpu.async_copy(
            outb_v.at[b], out_hbm.at[pl.ds(pbase + j * (CH // 2), CH // 2)],
            wsem[b])

    def wait_write(b):
        pltpu.make_async_copy(
            outb_v.at[b], out_hbm.at[pl.ds(pbase, CH // 2)], wsem[b]).wait()

    def select(j, b):
        rows = rows_v.at[b]
        outb = outb_v.at[b]

        def grp(g, carry):
            pv = idx_v[j, pl.ds(g * 16, 16)] & 1
            for l in range(16):
                i = g * 16 + l
                q = g * 8 + l // 2
                p = pv[l]
                src0 = p * EMBED_DIM
                dst0 = (l % 2) * EMBED_DIM
                for k in range(EMBED_DIM // 16):
                    outb[q, pl.ds(dst0 + k * 16, 16)] = (
                        rows[i, pl.ds(src0 + k * 16, 16)])
            return carry

        lax.fori_loop(0, CH // 16, grp, 0)

    for b in range(NBUF):
        compute_hi(b, b)
        fire_gather(b)

    def group(t, carry):
        for b in range(NBUF):
            j = t * NBUF + b
            wait_gather(b)

            @pl.when(t > 0)
            def _():
                wait_write(b)

            select(j, b)
            fire_write(j, b)

            @pl.when(t < NGROUP - 1)
            def _():
                compute_hi(j + NBUF, b)
                fire_gather(b)

        return carry

    lax.fori_loop(0, NGROUP, group, 0)

    for b in range(NBUF):
        wait_write(b)


def _unpack_tc_body(in_ref, out_ref):
    a = in_ref[...]
    out_ref[:, 0, :] = a[:, :EMBED_DIM]
    out_ref[:, 1, :] = a[:, EMBED_DIM:]


_unpack_tc = pl.pallas_call(
    _unpack_tc_body,
    grid=(N_PAIR // UNP_K,),
    in_specs=[pl.BlockSpec((UNP_K, 2 * EMBED_DIM), lambda i: (i, 0))],
    out_specs=pl.BlockSpec((UNP_K, 2, EMBED_DIM), lambda i: (i, 0, 0)),
    out_shape=jax.ShapeDtypeStruct((N_PAIR, 2, EMBED_DIM), jnp.float32),
)


def kernel(x, embedding_weight):
    idx = x.reshape(NW, NCHUNK, CH)
    packed_tbl = _pack_tc(embedding_weight.reshape(V_PAIR, 2, EMBED_DIM))
    packed_out = _gather_kernel(idx, packed_tbl)
    out = _unpack_tc(packed_out)
    return out.reshape(BATCH, SEQ, EMBED_DIM)


# speedup vs baseline: 1.0102x; 1.0007x over previous
import functools

import jax
import jax.numpy as jnp
from jax import lax
from jax.experimental import pallas as pl
from jax.experimental.pallas import tpu as pltpu
from jax.experimental.pallas import tpu_sc as plsc

VOCAB = 1000000
EMBED_DIM = 64
BATCH = 4096
SEQ = 200

NC = 2
NS = 16
NW = NC * NS

N_ROWS = BATCH * SEQ
PER_W = N_ROWS // NW
CH = 128
NCHUNK = PER_W // CH
NBUF = 4
NGROUP = NCHUNK // NBUF

N_PAIR = N_ROWS // 2
V_PAIR = VOCAB // 2

PK_BLK = 1250
UNP_K = 1024


def _pack_tc_body(in_ref, out_ref):
    out_ref[:, :EMBED_DIM] = in_ref[:, 0, :]
    out_ref[:, EMBED_DIM:] = in_ref[:, 1, :]


_pack_tc = pl.pallas_call(
    _pack_tc_body,
    grid=(V_PAIR // (PK_BLK * 4),),
    in_specs=[pl.BlockSpec((PK_BLK * 4, 2, EMBED_DIM), lambda i: (i, 0, 0))],
    out_specs=pl.BlockSpec((PK_BLK * 4, 2 * EMBED_DIM), lambda i: (i, 0)),
    out_shape=jax.ShapeDtypeStruct((V_PAIR, 2 * EMBED_DIM), jnp.float32),
)


@functools.partial(
    pl.kernel,
    out_type=jax.ShapeDtypeStruct((N_PAIR, 2 * EMBED_DIM), jnp.float32),
    mesh=plsc.VectorSubcoreMesh(core_axis_name="c", subcore_axis_name="s"),
    scratch_types=[
        pltpu.VMEM((NCHUNK, CH), jnp.int32),
        pltpu.VMEM((NBUF, CH), jnp.int32),
        pltpu.VMEM((NBUF, CH, 2 * EMBED_DIM), jnp.float32),
        pltpu.VMEM((NBUF, CH // 2, 2 * EMBED_DIM), jnp.float32),
    ]
    + [pltpu.SemaphoreType.DMA] * NBUF
    + [pltpu.SemaphoreType.DMA] * NBUF,
    compiler_params=pltpu.CompilerParams(use_tc_tiling_on_sc=True),
)
def _gather_kernel(idx_hbm, table_hbm, out_hbm, idx_v, hi_v, rows_v, outb_v,
                   *sems):
    gsem = sems[:NBUF]
    wsem = sems[NBUF:]
    wid = lax.axis_index("s") * NC + lax.axis_index("c")
    pbase = wid * (PER_W // 2)
    pltpu.sync_copy(idx_hbm.at[wid], idx_v)

    def compute_hi(j, b):
        for m in range(CH // 16):
            hi_v[b, pl.ds(m * 16, 16)] = lax.shift_right_logical(
                idx_v[j, pl.ds(m * 16, 16)], 1)

    def fire_gather(b):
        pltpu.async_copy(table_hbm.at[hi_v.at[b]], rows_v.at[b], gsem[b])

    def wait_gather(b):
        pltpu.make_async_copy(
            table_hbm.at[hi_v.at[b]], rows_v.at[b], gsem[b]).wait()

    def fire_write(j, b):
        pltpu.async_copy(
            outb_v.at[b], out_hbm.at[pl.ds(pbase + j * (CH // 2), CH // 2)],
            wsem[b])

    def wait_write(b):
        pltpu.make_async_copy(
            outb_v.at[b], out_hbm.at[pl.ds(pbase, CH // 2)], wsem[b]).wait()

    def select(j, b):
        rows = rows_v.at[b]
        outb = outb_v.at[b]

        def grp(g, carry):
            pv = idx_v[j, pl.ds(g * 16, 16)] & 1
            for l in range(16):
                i = g * 16 + l
                q = g * 8 + l // 2
                p = pv[l]
                src0 = p * EMBED_DIM
                dst0 = (l % 2) * EMBED_DIM
                for k in range(EMBED_DIM // 16):
                    outb[q, pl.ds(dst0 + k * 16, 16)] = (
                        rows[i, pl.ds(src0 + k * 16, 16)])
            return carry

        lax.fori_loop(0, CH // 16, grp, 0)

    for b in range(NBUF):
        compute_hi(b, b)
        fire_gather(b)

    def group(t, carry):
        for b in range(NBUF):
            j = t * NBUF + b
            wait_gather(b)

            @pl.when(t > 0)
            def _():
                wait_write(b)

            select(j, b)
            fire_write(j, b)

            @pl.when(t < NGROUP - 1)
            def _():
                compute_hi(j + NBUF, b)
                fire_gather(b)

        return carry

    lax.fori_loop(0, NGROUP, group, 0)

    for b in range(NBUF):
        wait_write(b)


def _unpack_tc_body(in_ref, out_ref):
    a = in_ref[...]
    out_ref[:, 0, :] = a[:, :EMBED_DIM]
    out_ref[:, 1, :] = a[:, EMBED_DIM:]


_unpack_tc = pl.pallas_call(
    _unpack_tc_body,
    grid=(N_PAIR // UNP_K,),
    in_specs=[pl.BlockSpec((UNP_K, 2 * EMBED_DIM), lambda i: (i, 0))],
    out_specs=pl.BlockSpec((UNP_K, 2, EMBED_DIM), lambda i: (i, 0, 0)),
    out_shape=jax.ShapeDtypeStruct((N_PAIR, 2, EMBED_DIM), jnp.float32),
)


def kernel(x, embedding_weight):
    idx = x.reshape(NW, NCHUNK, CH)
    packed_tbl = _pack_tc(embedding_weight.reshape(V_PAIR, 2, EMBED_DIM))
    packed_out = _gather_kernel(idx, packed_tbl)
    out = _unpack_tc(packed_out)
    return out.reshape(BATCH, SEQ, EMBED_DIM)


# speedup vs baseline: 1.4043x; 1.3902x over previous
import functools

import jax
import jax.numpy as jnp
from jax import lax
from jax.experimental import pallas as pl
from jax.experimental.pallas import tpu as pltpu
from jax.experimental.pallas import tpu_sc as plsc

VOCAB = 1000000
EMBED_DIM = 64
BATCH = 4096
SEQ = 200

NC = 2
NS = 16
NW = NC * NS

N_ROWS = BATCH * SEQ
PER_W = N_ROWS // NW
CH = 128
NCHUNK = PER_W // CH
NBUF = 4
NWB = 2

V_PAIR = VOCAB // 2

PK_BLK = 5000


def _pack_tc_body(top_ref, bot_ref, out_ref):
    out_ref[:, :EMBED_DIM] = top_ref[...]
    out_ref[:, EMBED_DIM:] = bot_ref[...]


_pack_tc = pl.pallas_call(
    _pack_tc_body,
    grid=(V_PAIR // PK_BLK,),
    in_specs=[
        pl.BlockSpec((PK_BLK, EMBED_DIM), lambda i: (i, 0)),
        pl.BlockSpec((PK_BLK, EMBED_DIM), lambda i: (i + V_PAIR // PK_BLK, 0)),
    ],
    out_specs=pl.BlockSpec((PK_BLK, 2 * EMBED_DIM), lambda i: (i, 0)),
    out_shape=jax.ShapeDtypeStruct((V_PAIR, 2 * EMBED_DIM), jnp.float32),
)


@functools.partial(
    pl.kernel,
    out_type=jax.ShapeDtypeStruct((N_ROWS, 2 * EMBED_DIM), jnp.float32),
    mesh=plsc.VectorSubcoreMesh(core_axis_name="c", subcore_axis_name="s"),
    scratch_types=[
        pltpu.VMEM((NCHUNK, CH), jnp.int32),
        pltpu.VMEM((NBUF, CH), jnp.int32),
        pltpu.VMEM((NBUF, CH, 2 * EMBED_DIM), jnp.float32),
        pltpu.VMEM((NWB, CH, 2 * EMBED_DIM), jnp.float32),
    ]
    + [pltpu.SemaphoreType.DMA] * NBUF
    + [pltpu.SemaphoreType.DMA] * NWB,
    compiler_params=pltpu.CompilerParams(use_tc_tiling_on_sc=True),
)
def _gather_kernel(idx_hbm, table_hbm, out_hbm, idx_v, hi_v, rows_v, outb_v,
                   *sems):
    gsem = sems[:NBUF]
    wsem = sems[NBUF:]
    wid = lax.axis_index("s") * NC + lax.axis_index("c")
    base = wid * PER_W
    pltpu.sync_copy(idx_hbm.at[wid], idx_v)

    def compute_hi(j, b):
        for m in range(CH // 16):
            v = idx_v[j, pl.ds(m * 16, 16)]
            wrap = lax.shift_right_logical(V_PAIR - 1 - v, 31)
            hi_v[b, pl.ds(m * 16, 16)] = v - wrap * V_PAIR

    def fire_gather(b):
        pltpu.async_copy(table_hbm.at[hi_v.at[b]], rows_v.at[b], gsem[b])

    def wait_gather(b):
        pltpu.make_async_copy(
            table_hbm.at[hi_v.at[b]], rows_v.at[b], gsem[b]).wait()

    def fire_write(j, w):
        pltpu.async_copy(
            outb_v.at[w], out_hbm.at[pl.ds(base + j * CH, CH)], wsem[w])

    def wait_write(w):
        pltpu.make_async_copy(
            outb_v.at[w], out_hbm.at[pl.ds(base, CH)], wsem[w]).wait()

    def select(j, b, w):
        rows = rows_v.at[b]
        outb = outb_v.at[w]

        def grp(g, carry):
            pv = lax.shift_right_logical(
                V_PAIR - 1 - idx_v[j, pl.ds(g * 16, 16)], 31)
            for l in range(16):
                i = g * 16 + l
                p = pv[l]
                src0 = p * EMBED_DIM
                for k in range(EMBED_DIM // 16):
                    outb[i, pl.ds(k * 16, 16)] = (
                        rows[i, pl.ds(src0 + k * 16, 16)])
            return carry

        lax.fori_loop(0, CH // 16, grp, 0)

    for b in range(NBUF):
        compute_hi(b, b)
        fire_gather(b)

    def group(t, carry):
        for b in range(NBUF):
            j = t * NBUF + b
            w = b % NWB
            wait_gather(b)

            if b >= NWB:
                wait_write(w)
            else:
                @pl.when(t > 0)
                def _():
                    wait_write(w)

            select(j, b, w)
            fire_write(j, w)

            @pl.when(t < NCHUNK // NBUF - 1)
            def _():
                compute_hi(j + NBUF, b)
                fire_gather(b)

        return carry

    lax.fori_loop(0, NCHUNK // NBUF, group, 0)

    for w in range(NWB):
        wait_write(w)


def kernel(x, embedding_weight):
    idx = x.reshape(NW, NCHUNK, CH)
    packed_tbl = _pack_tc(embedding_weight, embedding_weight)
    wide = _gather_kernel(idx, packed_tbl)
    return wide[:, :EMBED_DIM].reshape(BATCH, SEQ, EMBED_DIM)
